# baseline (speedup 1.0000x reference)
import functools
import math

import jax
import jax.numpy as jnp
from jax import lax
from jax.experimental import pallas as pl
from jax.experimental.pallas import tpu as pltpu
from jax.experimental.pallas import tpu_sc as plsc

D = 64
SCALE = math.sqrt(D)
NBUF = 2
ROWS = 128
VBLK = 128


def _make_relayout(V: int):
  info = plsc.get_sparse_core_info()
  NC, NS, L = info.num_cores, info.num_subcores, info.num_lanes
  NW = NC * NS
  n_full = V // VBLK
  n_iter = (n_full + NW - 1) // NW

  mesh = plsc.VectorSubcoreMesh(core_axis_name="c", subcore_axis_name="s")

  @functools.partial(
      pl.kernel,
      out_type=jax.ShapeDtypeStruct((V * D // 128, 128), jnp.float32),
      mesh=mesh,
      scratch_types=[
          pltpu.VMEM((NBUF, D, VBLK), jnp.float32),
          pltpu.VMEM((NBUF, D, 131), jnp.float32),
          pltpu.SemaphoreType.DMA,
          pltpu.SemaphoreType.DMA,
          pltpu.SemaphoreType.DMA,
          pltpu.SemaphoreType.DMA,
      ],
      compiler_params=pltpu.CompilerParams(
          use_tc_tiling_on_sc=True, needs_layout_passes=False),
  )
  def relayout(tt_hbm, out_hbm, tiles_v, cbuf, g0, g1, s0, s1):
    gsem = (g0, g1)
    ssem = (s0, s1)
    wid = lax.axis_index("s") * NC + lax.axis_index("c")

    def load(c, b):
      return pltpu.make_async_copy(
          tt_hbm.at[:, pl.ds(c * VBLK, VBLK)], tiles_v.at[b], gsem[b])

    def store(c, b):
      return pltpu.make_async_copy(
          cbuf.at[b, :, pl.ds(0, 128)],
          out_hbm.at[pl.ds(c * (VBLK * D // 128), VBLK * D // 128)],
          ssem[b])

    lane = lax.iota(jnp.int32, L)
    r2_ids = [(lane + vg * L) >> 1 for vg in range(VBLK // L)]
    j_base = [((lane + vg * L) & 1) * D for vg in range(VBLK // L)]

    for b in range(NBUF):
      @pl.when(wid + b * NW < n_full)
      def _():
        load(wid + b * NW, b).start()

    @pl.loop(0, n_iter, step=NBUF)
    def _(t):
      for b in range(NBUF):
        c = wid + (t + b) * NW

        @pl.when(c < n_full)
        def _():
          load(c, b).wait()

          @pl.when(c >= NW * NBUF)
          def _():
            store(c - NW * NBUF, b).wait()

          tv = tiles_v.at[b]
          cb = cbuf.at[b]

          @plsc.parallel_loop(0, D, unroll=2)
          def _(f):
            colf = jnp.full((L,), f, jnp.int32)
            for vg in range(VBLK // L):
              v = tv[f, pl.ds(vg * L, L)]
              plsc.store_scatter(cb, [r2_ids[vg], j_base[vg] + colf], v)

          nc = c + NW * NBUF

          @pl.when(nc < n_full)
          def _():
            load(nc, b).start()

          store(c, b).start()

    for b in range(NBUF):
      store(0, b).wait()

  return relayout


def _make_lookup(S0: int, S1: int, V2: int):
  info = plsc.get_sparse_core_info()
  NC, NS, L = info.num_cores, info.num_subcores, info.num_lanes
  NW = NC * NS
  n_mblk = S0 // ROWS
  n_groups = S1 * n_mblk
  assert n_groups % NW == 0
  g_per_w = n_groups // NW
  assert g_per_w % NBUF == 0
  d_blk = D // 8

  mesh = plsc.VectorSubcoreMesh(core_axis_name="c", subcore_axis_name="s")

  @functools.partial(
      pl.kernel,
      out_type=jax.ShapeDtypeStruct((S1, d_blk, n_mblk, 8, L * 8), jnp.float32),
      mesh=mesh,
      scratch_types=[
          pltpu.VMEM((g_per_w * ROWS,), jnp.int32),
          pltpu.VMEM((NBUF, 2 * ROWS, D // 2), jnp.float32),
          pltpu.VMEM((NBUF, 2 * ROWS), jnp.int32),
          pltpu.VMEM((NBUF, D, L * 8 + 1), jnp.float32),
          pltpu.SemaphoreType.DMA,
          pltpu.SemaphoreType.DMA,
          pltpu.SemaphoreType.DMA,
          pltpu.SemaphoreType.DMA,
      ],
      compiler_params=pltpu.CompilerParams(
          use_tc_tiling_on_sc=False, needs_layout_passes=False),
  )
  def lookup(idx_hbm, t32_hbm, out_hbm, idx_v, rows_v, iv, tbuf,
             g0, g1, s0, s1):
    gsem = (g0, g1)
    ssem = (s0, s1)
    wid = lax.axis_index("s") * NC + lax.axis_index("c")
    base = wid * g_per_w
    pltpu.sync_copy(idx_hbm.at[pl.ds(base * ROWS, g_per_w * ROWS)], idx_v)

    def fill_iv(gl, b):
      for icg in range(ROWS // L):
        a = idx_v[pl.ds(gl * ROWS + icg * L, L)]
        a2 = a + a
        iv[b, pl.ds(icg * L, L)] = a2
        iv[b, pl.ds(ROWS + icg * L, L)] = a2 + 1

    def gathers(b):
      return [
          pltpu.make_async_copy(
              t32_hbm.at[iv.at[b, pl.ds(h * ROWS, ROWS)]],
              rows_v.at[b, pl.ds(h * ROWS, ROWS)], gsem[b])
          for h in range(2)
      ]

    def stores(gl, b):
      gid = base + gl
      j = gid >> 5
      m = gid & (n_mblk - 1)
      return [
          pltpu.make_async_copy(
              tbuf.at[b, pl.ds(8 * k, 8), pl.ds(0, L * 8)],
              out_hbm.at[j, k, m], ssem[b])
          for k in range(d_blk)
      ]

    lane = lax.iota(jnp.int32, L)
    f_ids = [lane + (fg * L) for fg in range(D // L)]

    for b in range(NBUF):
      fill_iv(b, b)
      for d in gathers(b):
        d.start()

    @pl.loop(0, g_per_w, step=NBUF)
    def _(g):
      for b in range(NBUF):
        gl = g + b
        for d in gathers(b):
          d.wait()

        @pl.when(gl >= NBUF)
        def _():
          for d in stores(gl - NBUF, b):
            d.wait()

        tb = tbuf.at[b]

        @plsc.parallel_loop(0, ROWS, unroll=4)
        def _(ic):
          col = jnp.full((L,), ic, jnp.int32)
          for fg in range(D // L):
            v = rows_v[b, ic + (fg // 2) * ROWS, pl.ds((fg % 2) * L, L)]
            plsc.store_scatter(tb, [f_ids[fg], col], v * SCALE)

        nxt = gl + NBUF

        @pl.when(nxt < g_per_w)
        def _():
          fill_iv(nxt, b)
          for d in gathers(b):
            d.start()

        for d in stores(gl, b):
          d.start()

    for b in range(NBUF):
      for d in stores(g_per_w - NBUF + b, b):
        d.wait()

  return lookup


def kernel(x, table):
  S0, S1 = x.shape
  V, _ = table.shape
  n_full = V // VBLK
  lin = _make_relayout(V)(table.T)
  tail = table[n_full * VBLK:].reshape(-1, 128)
  lin = lax.dynamic_update_slice(lin, tail, (n_full * VBLK * D // 128, 0))
  t32 = lin.reshape(V * 2, D // 2)
  idx = x.T.reshape(S0 * S1).astype(jnp.int32)
  arr = _make_lookup(S0, S1, V * 2)(idx, t32)
  return arr.transpose((2, 4, 0, 1, 3)).reshape(S0, S1, D)


# speedup vs baseline: 2.6114x; 2.6114x over previous
import functools
import math

import jax
import jax.numpy as jnp
from jax import lax
from jax.experimental import pallas as pl
from jax.experimental.pallas import tpu as pltpu
from jax.experimental.pallas import tpu_sc as plsc

D = 64
SCALE = math.sqrt(D)
NBUF = 2
ROWS = 128
VBLK = 128


def _make_relayout(V: int):
  info = plsc.get_sparse_core_info()
  NC, NS, L = info.num_cores, info.num_subcores, info.num_lanes
  NW = NC * NS
  n_full = V // VBLK
  n_iter = (n_full + NW - 1) // NW

  mesh = plsc.VectorSubcoreMesh(core_axis_name="c", subcore_axis_name="s")

  @functools.partial(
      pl.kernel,
      out_type=jax.ShapeDtypeStruct((V * D,), jnp.float32),
      mesh=mesh,
      scratch_types=[
          pltpu.VMEM((D, VBLK), jnp.float32),
          pltpu.VMEM((D, VBLK), jnp.float32),
          pltpu.VMEM((8576,), jnp.float32),
          pltpu.VMEM((8576,), jnp.float32),
          pltpu.VMEM((D * VBLK,), jnp.float32),
          pltpu.VMEM((D * VBLK,), jnp.float32),
          pltpu.SemaphoreType.DMA,
          pltpu.SemaphoreType.DMA,
          pltpu.SemaphoreType.DMA,
          pltpu.SemaphoreType.DMA,
      ],
      compiler_params=pltpu.CompilerParams(
          use_tc_tiling_on_sc=True, needs_layout_passes=False),
  )
  def relayout(tt_hbm, out_hbm, tiles0, tiles1, tflat0, tflat1,
               cb0, cb1, g0, g1, s0, s1):
    tiles_b = (tiles0, tiles1)
    tflat_b = (tflat0, tflat1)
    cb_b = (cb0, cb1)
    gsem = (g0, g1)
    ssem = (s0, s1)
    wid = lax.axis_index("s") * NC + lax.axis_index("c")

    def load(c, b):
      return pltpu.make_async_copy(
          tt_hbm.at[:, pl.ds(c * VBLK, VBLK)], tiles_b[b], gsem[b])

    def store(c, b):
      return pltpu.make_async_copy(
          cb_b[b], out_hbm.at[pl.ds(c * (VBLK * D), VBLK * D)], ssem[b])

    lane = lax.iota(jnp.int32, L)
    f133 = [(lane + fg * L) * 133 for fg in range(D // L)]

    for b in range(NBUF):
      @pl.when(wid + b * NW < n_full)
      def _():
        load(wid + b * NW, b).start()

    @pl.loop(0, n_iter, step=NBUF)
    def _(t):
      for b in range(NBUF):
        c = wid + (t + b) * NW

        @pl.when(c < n_full)
        def _():
          load(c, b).wait()
          tf = tflat_b[b]
          tv = tiles_b[b]

          @plsc.parallel_loop(0, D, unroll=2)
          def _(f):
            for vg in range(VBLK // L):
              tf[pl.ds(f * 133 + vg * L, L)] = tv[f, pl.ds(vg * L, L)]

          nc = c + NW * NBUF

          @pl.when(nc < n_full)
          def _():
            load(nc, b).start()

          @pl.when(c >= NW * NBUF)
          def _():
            store(c - NW * NBUF, b).wait()

          cb = cb_b[b]

          @plsc.parallel_loop(0, VBLK, unroll=2)
          def _(v):
            for fg in range(D // L):
              vals = plsc.load_gather(tf, [f133[fg] + v])
              cb[pl.ds(v * D + fg * L, L)] = vals

          store(c, b).start()

    for b in range(NBUF):
      store(0, b).wait()

  return relayout


def _make_lookup(S0: int, S1: int, V2: int):
  info = plsc.get_sparse_core_info()
  NC, NS, L = info.num_cores, info.num_subcores, info.num_lanes
  NW = NC * NS
  n_mblk = S0 // ROWS
  n_groups = S1 * n_mblk
  assert n_groups % NW == 0
  g_per_w = n_groups // NW
  assert g_per_w % NBUF == 0
  d_blk = D // 8

  mesh = plsc.VectorSubcoreMesh(core_axis_name="c", subcore_axis_name="s")

  @functools.partial(
      pl.kernel,
      out_type=jax.ShapeDtypeStruct((S1, d_blk, n_mblk, 8, L * 8), jnp.float32),
      mesh=mesh,
      scratch_types=[
          pltpu.VMEM((g_per_w * ROWS,), jnp.int32),
          pltpu.VMEM((NBUF, 2 * ROWS, D // 2), jnp.float32),
          pltpu.VMEM((NBUF, 2 * ROWS), jnp.int32),
          pltpu.VMEM((NBUF, D, L * 8 + 1), jnp.float32),
          pltpu.SemaphoreType.DMA,
          pltpu.SemaphoreType.DMA,
          pltpu.SemaphoreType.DMA,
          pltpu.SemaphoreType.DMA,
      ],
      compiler_params=pltpu.CompilerParams(
          use_tc_tiling_on_sc=False, needs_layout_passes=False),
  )
  def lookup(idx_hbm, t32_hbm, out_hbm, idx_v, rows_v, iv, tbuf,
             g0, g1, s0, s1):
    gsem = (g0, g1)
    ssem = (s0, s1)
    wid = lax.axis_index("s") * NC + lax.axis_index("c")
    base = wid * g_per_w
    pltpu.sync_copy(idx_hbm.at[pl.ds(base * ROWS, g_per_w * ROWS)], idx_v)

    def fill_iv(gl, b):
      for icg in range(ROWS // L):
        a = idx_v[pl.ds(gl * ROWS + icg * L, L)]
        a2 = a + a
        iv[b, pl.ds(icg * L, L)] = a2
        iv[b, pl.ds(ROWS + icg * L, L)] = a2 + 1

    def gathers(b):
      return [
          pltpu.make_async_copy(
              t32_hbm.at[iv.at[b, pl.ds(h * ROWS, ROWS)]],
              rows_v.at[b, pl.ds(h * ROWS, ROWS)], gsem[b])
          for h in range(2)
      ]

    def stores(gl, b):
      gid = base + gl
      j = gid >> 5
      m = gid & (n_mblk - 1)
      return [
          pltpu.make_async_copy(
              tbuf.at[b, pl.ds(8 * k, 8), pl.ds(0, L * 8)],
              out_hbm.at[j, k, m], ssem[b])
          for k in range(d_blk)
      ]

    lane = lax.iota(jnp.int32, L)
    f_ids = [lane + (fg * L) for fg in range(D // L)]

    for b in range(NBUF):
      fill_iv(b, b)
      for d in gathers(b):
        d.start()

    @pl.loop(0, g_per_w, step=NBUF)
    def _(g):
      for b in range(NBUF):
        gl = g + b
        for d in gathers(b):
          d.wait()

        @pl.when(gl >= NBUF)
        def _():
          for d in stores(gl - NBUF, b):
            d.wait()

        tb = tbuf.at[b]

        @plsc.parallel_loop(0, ROWS, unroll=4)
        def _(ic):
          col = jnp.full((L,), ic, jnp.int32)
          for fg in range(D // L):
            v = rows_v[b, ic + (fg // 2) * ROWS, pl.ds((fg % 2) * L, L)]
            plsc.store_scatter(tb, [f_ids[fg], col], v * SCALE)

        nxt = gl + NBUF

        @pl.when(nxt < g_per_w)
        def _():
          fill_iv(nxt, b)
          for d in gathers(b):
            d.start()

        for d in stores(gl, b):
          d.start()

    for b in range(NBUF):
      for d in stores(g_per_w - NBUF + b, b):
        d.wait()

  return lookup


def kernel(x, table):
  S0, S1 = x.shape
  V, _ = table.shape
  n_full = V // VBLK
  lin = _make_relayout(V)(table.T)
  tail = table[n_full * VBLK:].reshape(-1)
  lin = lax.dynamic_update_slice(lin, tail, (n_full * VBLK * D,))
  t32 = lin.reshape(V * 2, D // 2)
  idx = x.T.reshape(S0 * S1).astype(jnp.int32)
  arr = _make_lookup(S0, S1, V * 2)(idx, t32)
  return arr.transpose((2, 4, 0, 1, 3)).reshape(S0, S1, D)
